# baseline (speedup 1.0000x reference)
import functools
import math

import jax
import jax.numpy as jnp
from jax import lax
from jax.experimental import pallas as pl
from jax.experimental.pallas import tpu as pltpu
from jax.experimental.pallas import tpu_sc as plsc

_D = 64
_B = 4096
_L = 200
_V = 1000000
_N = _B * _L
_NC = 2
_NS = 16
_NW = _NC * _NS
_SCALE = math.sqrt(_D)

_VT = _V // 128
_VMAIN = _VT * 128
_TPB = 2
_QT = _VT // _TPB
_Q_PER_W = _QT // _NW
_Q_EXTRA = _QT - _Q_PER_W * _NW

_LPB = 2
_NBLK = _L // _LPB


def _iota16():
    return lax.iota(jnp.int32, 16)


def _k1_body(wt, wtail, tbl, in_v, out_v, tail_v, sg0, sg1, sw):
    wid = lax.axis_index("s") * _NC + lax.axis_index("c")
    start = wid * _Q_PER_W + jnp.minimum(wid, _Q_EXTRA)
    cnt = _Q_PER_W + jnp.where(wid < _Q_EXTRA, 1, 0)
    sg = (sg0, sg1)

    perm = []
    for t in range(_TPB):
        for j in range(8):
            v = j * 16 + _iota16()
            perm.append((t * 128 + j * 16, t * 64 + v // 2, (v % 2) * 64))

    def g_issue(q, b):
        for i in range(8):
            pltpu.async_copy(
                wt.at[pl.ds(i * 8, 8), pl.ds(q * (128 * _TPB), 128 * _TPB)],
                in_v.at[b, pl.ds(i * 8, 8)],
                sg[b],
            )

    def g_wait(b):
        pltpu.make_async_copy(
            wt.at[:, pl.ds(0, 128 * _TPB)], in_v.at[b], sg[b]
        ).wait()

    def w_issue(q, b):
        pltpu.async_copy(
            out_v.at[b, :, pl.ds(0, 128)],
            tbl.at[pl.ds(q * (64 * _TPB), 64 * _TPB)],
            sw,
        )

    def w_wait(b):
        pltpu.make_async_copy(
            tbl.at[pl.ds(0, 64 * _TPB)], out_v.at[b, :, pl.ds(0, 128)], sw
        ).wait()

    g_issue(start, 0)

    @pl.loop(0, (_Q_PER_W + 1 + 1) // 2 + 1)
    def _pairk(p):
        for b in range(2):
            k = p * 2 + b

            @pl.when(k < cnt)
            def _(b=b, k=k):
                q = start + k
                g_wait(b)

                @pl.when(k + 1 < cnt)
                def _():
                    g_issue(q + 1, 1 - b)

                @pl.when(k >= 2)
                def _():
                    w_wait(b)

                @plsc.parallel_loop(0, 64, 1, unroll=4)
                def _tp(c, _b=b):
                    for off, d0, d1 in perm:
                        vals = in_v[_b, c, pl.ds(off, 16)] * _SCALE
                        plsc.store_scatter(out_v.at[_b], [d0, d1 + c], vals)

                w_issue(q, b)

    for i in range(2):
        @pl.when(cnt > i)
        def _(i=i):
            w_wait((cnt - 2 + i) % 2)

    @pl.when(wid == _NW - 1)
    def _tail():
        pltpu.sync_copy(wtail, tail_v)

        @plsc.parallel_loop(0, 64, 1, unroll=4)
        def _tp(c):
            for j in range(4):
                v = j * 16 + _iota16()
                vals = tail_v[c, pl.ds(j * 16, 16)] * _SCALE
                plsc.store_scatter(
                    out_v.at[0], [v // 2, (v % 2) * 64 + c], vals
                )

        pltpu.sync_copy(
            out_v.at[0, pl.ds(0, 32), pl.ds(0, 128)],
            tbl.at[pl.ds(_VMAIN // 2, 32)],
        )


@functools.partial(
    pl.kernel,
    out_type=jax.ShapeDtypeStruct((_V // 2, 128), jnp.float32),
    mesh=plsc.VectorSubcoreMesh(core_axis_name="c", subcore_axis_name="s"),
    scratch_types=[
        pltpu.VMEM((2, _D, 128 * _TPB), jnp.float32),
        pltpu.VMEM((2, 64 * _TPB, 137), jnp.float32),
        pltpu.VMEM((_D, _D), jnp.float32),
        pltpu.SemaphoreType.DMA,
        pltpu.SemaphoreType.DMA,
        pltpu.SemaphoreType.DMA,
    ],
    compiler_params=pltpu.CompilerParams(
        use_tc_tiling_on_sc=True, needs_layout_passes=False
    ),
)
def _repack(wt, wtail, tbl, in_v, out_v, tail_v, sg0, sg1, sw):
    _k1_body(wt, wtail, tbl, in_v, out_v, tail_v, sg0, sg1, sw)


def _k2_body(tbl, idxt, out, idx_v, rows_v, out_v,
             sg0, sg1, sg2, sg3, si0, si1, si2, si3, sw):
    wid = lax.axis_index("s") * _NC + lax.axis_index("c")
    sg = (sg0, sg1, sg2, sg3)
    si = (si0, si1, si2, si3)

    perm = []
    for u in range(_LPB):
        pu = []
        for j in range(4):
            c = j * 16 + _iota16()
            pu.append((u * 8 + c // 8, c % 8))
        perm.append(pu)

    def i_issue(blk, b):
        pltpu.async_copy(
            idxt.at[pl.ds(blk * _LPB, _LPB), pl.ds(wid * 128, 128)],
            idx_v.at[b], si[b],
        )

    def i_wait(blk, b):
        pltpu.make_async_copy(
            idxt.at[pl.ds(blk * _LPB, _LPB), pl.ds(wid * 128, 128)],
            idx_v.at[b], si[b],
        ).wait()

    def g_issue(b):
        for u in range(_LPB):
            pltpu.async_copy(
                tbl.at[idx_v.at[b, u]],
                rows_v.at[b, pl.ds(u * 128, 128)],
                sg[b],
            )

    def g_wait(b):
        pltpu.make_async_copy(tbl.at[pl.ds(0, _LPB * 128)], rows_v.at[b], sg[b]).wait()

    def w_issue(blk, b):
        l0 = blk * _LPB
        for u in range(_LPB):
            pltpu.async_copy(
                out_v.at[b, pl.ds(u * 8, 8), :, pl.ds(0, 128)],
                out.at[l0 + u, :, wid],
                sw,
            )

    def w_wait(b):
        for u in range(_LPB):
            pltpu.make_async_copy(
                out.at[0, :, 0],
                out_v.at[b, pl.ds(u * 8, 8), :, pl.ds(0, 128)],
                sw,
            ).wait()

    for i in range(3):
        i_issue(jnp.int32(i), i)
    i_wait(jnp.int32(0), 0)
    g_issue(0)
    i_wait(jnp.int32(1), 1)
    g_issue(1)

    @pl.loop(0, _NBLK // 4)
    def _quad(p):
        for b in range(4):
            blk = p * 4 + b

            g_wait(b)

            @pl.when(blk + 3 < _NBLK)
            def _(b=b, blk=blk):
                i_issue(blk + 3, (b + 3) % 4)

            @pl.when(blk + 2 < _NBLK)
            def _(b=b, blk=blk):
                i_wait(blk + 2, (b + 2) % 4)
                g_issue((b + 2) % 4)

            @pl.when(blk >= 2)
            def _(b=b):
                w_wait(b % 2)

            for u in range(_LPB):

                @plsc.parallel_loop(0, 128, 1, unroll=4)
                def _tp(rr, _b=b, _u=u):
                    for jj in range(4):
                        d0, d1 = perm[_u][jj]
                        vals = rows_v[_b, _u * 128 + rr, pl.ds(jj * 16, 16)]
                        plsc.store_scatter(
                            out_v.at[_b % 2],
                            [d0, d1, jnp.zeros((16,), jnp.int32) + rr],
                            vals,
                        )

            w_issue(blk, b % 2)

    @pl.loop(0, 1)
    def _fin(_):
        for blkf in (_NBLK - 2, _NBLK - 1):
            w_wait(blkf % 2)


@functools.partial(
    pl.kernel,
    out_type=jax.ShapeDtypeStruct((_L, 8, _NW, 8, 128), jnp.float32),
    mesh=plsc.VectorSubcoreMesh(core_axis_name="c", subcore_axis_name="s"),
    scratch_types=[
        pltpu.VMEM((4, _LPB, 128), jnp.int32),
        pltpu.VMEM((4, _LPB * 128, _D), jnp.float32),
        pltpu.VMEM((2, _LPB * 8, 8, 137), jnp.float32),
        pltpu.SemaphoreType.DMA,
        pltpu.SemaphoreType.DMA,
        pltpu.SemaphoreType.DMA,
        pltpu.SemaphoreType.DMA,
        pltpu.SemaphoreType.DMA,
        pltpu.SemaphoreType.DMA,
        pltpu.SemaphoreType.DMA,
        pltpu.SemaphoreType.DMA,
        pltpu.SemaphoreType.DMA,
    ],
    compiler_params=pltpu.CompilerParams(
        use_tc_tiling_on_sc=False, needs_layout_passes=False
    ),
)
def _gather(tbl, idxt, out, idx_v, rows_v, out_v,
            sg0, sg1, sg2, sg3, si0, si1, si2, si3, sw):
    _k2_body(tbl, idxt, out, idx_v, rows_v, out_v,
             sg0, sg1, sg2, sg3, si0, si1, si2, si3, sw)


def kernel(x, embedding_weight):
    wt = embedding_weight.T
    wtail = embedding_weight[_VMAIN:].T
    tbl2 = _repack(wt, wtail)
    tbl = tbl2.reshape(_V, _D)
    idxt = x.astype(jnp.int32).T
    o5 = _gather(tbl, idxt)
    return o5.transpose(2, 4, 0, 1, 3).reshape(_B, _L, _D)


# speedup vs baseline: 1.0043x; 1.0043x over previous
import functools
import math

import jax
import jax.numpy as jnp
from jax import lax
from jax.experimental import pallas as pl
from jax.experimental.pallas import tpu as pltpu
from jax.experimental.pallas import tpu_sc as plsc

_D = 64
_B = 4096
_L = 200
_V = 1000000
_N = _B * _L
_NC = 2
_NS = 16
_NW = _NC * _NS
_SCALE = math.sqrt(_D)

_VT = _V // 128
_VMAIN = _VT * 128
_TPB = 2
_QT = _VT // _TPB
_Q_PER_W = _QT // _NW
_Q_EXTRA = _QT - _Q_PER_W * _NW

_LPB = 2
_NBLK = _L // _LPB


def _iota16():
    return lax.iota(jnp.int32, 16)


def _k1_body(wt, wtail, tbl, in_v, out_v, tail_v, sg0, sg1, sw):
    wid = lax.axis_index("s") * _NC + lax.axis_index("c")
    start = wid * _Q_PER_W + jnp.minimum(wid, _Q_EXTRA)
    cnt = _Q_PER_W + jnp.where(wid < _Q_EXTRA, 1, 0)
    sg = (sg0, sg1)

    perm = []
    for t in range(_TPB):
        for j in range(8):
            v = j * 16 + _iota16()
            perm.append((t * 128 + j * 16, t * 64 + v // 2, (v % 2) * 64))

    def g_issue(q, b):
        for i in range(8):
            pltpu.async_copy(
                wt.at[pl.ds(i * 8, 8), pl.ds(q * (128 * _TPB), 128 * _TPB)],
                in_v.at[b, pl.ds(i * 8, 8)],
                sg[b],
            )

    def g_wait(b):
        pltpu.make_async_copy(
            wt.at[:, pl.ds(0, 128 * _TPB)], in_v.at[b], sg[b]
        ).wait()

    def w_issue(q, b):
        pltpu.async_copy(
            out_v.at[b, :, pl.ds(0, 128)],
            tbl.at[pl.ds(q * (64 * _TPB), 64 * _TPB)],
            sw,
        )

    def w_wait(b):
        pltpu.make_async_copy(
            tbl.at[pl.ds(0, 64 * _TPB)], out_v.at[b, :, pl.ds(0, 128)], sw
        ).wait()

    g_issue(start, 0)

    @pl.loop(0, (_Q_PER_W + 1 + 1) // 2 + 1)
    def _pairk(p):
        for b in range(2):
            k = p * 2 + b

            @pl.when(k < cnt)
            def _(b=b, k=k):
                q = start + k
                g_wait(b)

                @pl.when(k + 1 < cnt)
                def _():
                    g_issue(q + 1, 1 - b)

                @pl.when(k >= 2)
                def _():
                    w_wait(b)

                @plsc.parallel_loop(0, 64, 1, unroll=2)
                def _tp(c, _b=b):
                    for off, d0, d1 in perm:
                        vals = in_v[_b, c, pl.ds(off, 16)] * _SCALE
                        plsc.store_scatter(out_v.at[_b], [d0, d1 + c], vals)

                w_issue(q, b)

    for i in range(2):
        @pl.when(cnt > i)
        def _(i=i):
            w_wait((cnt - 2 + i) % 2)

    @pl.when(wid == _NW - 1)
    def _tail():
        pltpu.sync_copy(wtail, tail_v)

        @plsc.parallel_loop(0, 64, 1, unroll=2)
        def _tp(c):
            for j in range(4):
                v = j * 16 + _iota16()
                vals = tail_v[c, pl.ds(j * 16, 16)] * _SCALE
                plsc.store_scatter(
                    out_v.at[0], [v // 2, (v % 2) * 64 + c], vals
                )

        pltpu.sync_copy(
            out_v.at[0, pl.ds(0, 32), pl.ds(0, 128)],
            tbl.at[pl.ds(_VMAIN // 2, 32)],
        )


@functools.partial(
    pl.kernel,
    out_type=jax.ShapeDtypeStruct((_V // 2, 128), jnp.float32),
    mesh=plsc.VectorSubcoreMesh(core_axis_name="c", subcore_axis_name="s"),
    scratch_types=[
        pltpu.VMEM((2, _D, 128 * _TPB), jnp.float32),
        pltpu.VMEM((2, 64 * _TPB, 137), jnp.float32),
        pltpu.VMEM((_D, _D), jnp.float32),
        pltpu.SemaphoreType.DMA,
        pltpu.SemaphoreType.DMA,
        pltpu.SemaphoreType.DMA,
    ],
    compiler_params=pltpu.CompilerParams(
        use_tc_tiling_on_sc=True, needs_layout_passes=False
    ),
)
def _repack(wt, wtail, tbl, in_v, out_v, tail_v, sg0, sg1, sw):
    _k1_body(wt, wtail, tbl, in_v, out_v, tail_v, sg0, sg1, sw)


def _k2_body(tbl, idxt, out, idx_v, rows_v, out_v,
             sg0, sg1, sg2, sg3, si0, si1, si2, si3, sw):
    wid = lax.axis_index("s") * _NC + lax.axis_index("c")
    sg = (sg0, sg1, sg2, sg3)
    si = (si0, si1, si2, si3)

    perm = []
    for u in range(_LPB):
        pu = []
        for j in range(4):
            c = j * 16 + _iota16()
            pu.append((u * 8 + c // 8, c % 8))
        perm.append(pu)

    def i_issue(blk, b):
        pltpu.async_copy(
            idxt.at[pl.ds(blk * _LPB, _LPB), pl.ds(wid * 128, 128)],
            idx_v.at[b], si[b],
        )

    def i_wait(blk, b):
        pltpu.make_async_copy(
            idxt.at[pl.ds(blk * _LPB, _LPB), pl.ds(wid * 128, 128)],
            idx_v.at[b], si[b],
        ).wait()

    def g_issue(b):
        for u in range(_LPB):
            pltpu.async_copy(
                tbl.at[idx_v.at[b, u]],
                rows_v.at[b, pl.ds(u * 128, 128)],
                sg[b],
            )

    def g_wait(b):
        pltpu.make_async_copy(tbl.at[pl.ds(0, _LPB * 128)], rows_v.at[b], sg[b]).wait()

    def w_issue(blk, b):
        l0 = blk * _LPB
        for u in range(_LPB):
            pltpu.async_copy(
                out_v.at[b, pl.ds(u * 8, 8), :, pl.ds(0, 128)],
                out.at[l0 + u, :, wid],
                sw,
            )

    def w_wait(b):
        for u in range(_LPB):
            pltpu.make_async_copy(
                out.at[0, :, 0],
                out_v.at[b, pl.ds(u * 8, 8), :, pl.ds(0, 128)],
                sw,
            ).wait()

    for i in range(3):
        i_issue(jnp.int32(i), i)
    i_wait(jnp.int32(0), 0)
    g_issue(0)
    i_wait(jnp.int32(1), 1)
    g_issue(1)

    @pl.loop(0, _NBLK // 4)
    def _quad(p):
        for b in range(4):
            blk = p * 4 + b

            g_wait(b)

            @pl.when(blk + 3 < _NBLK)
            def _(b=b, blk=blk):
                i_issue(blk + 3, (b + 3) % 4)

            @pl.when(blk + 2 < _NBLK)
            def _(b=b, blk=blk):
                i_wait(blk + 2, (b + 2) % 4)
                g_issue((b + 2) % 4)

            @pl.when(blk >= 2)
            def _(b=b):
                w_wait(b % 2)

            for u in range(_LPB):

                @plsc.parallel_loop(0, 128, 1, unroll=2)
                def _tp(rr, _b=b, _u=u):
                    for jj in range(4):
                        d0, d1 = perm[_u][jj]
                        vals = rows_v[_b, _u * 128 + rr, pl.ds(jj * 16, 16)]
                        plsc.store_scatter(
                            out_v.at[_b % 2],
                            [d0, d1, jnp.zeros((16,), jnp.int32) + rr],
                            vals,
                        )

            w_issue(blk, b % 2)

    @pl.loop(0, 1)
    def _fin(_):
        for blkf in (_NBLK - 2, _NBLK - 1):
            w_wait(blkf % 2)


@functools.partial(
    pl.kernel,
    out_type=jax.ShapeDtypeStruct((_L, 8, _NW, 8, 128), jnp.float32),
    mesh=plsc.VectorSubcoreMesh(core_axis_name="c", subcore_axis_name="s"),
    scratch_types=[
        pltpu.VMEM((4, _LPB, 128), jnp.int32),
        pltpu.VMEM((4, _LPB * 128, _D), jnp.float32),
        pltpu.VMEM((2, _LPB * 8, 8, 137), jnp.float32),
        pltpu.SemaphoreType.DMA,
        pltpu.SemaphoreType.DMA,
        pltpu.SemaphoreType.DMA,
        pltpu.SemaphoreType.DMA,
        pltpu.SemaphoreType.DMA,
        pltpu.SemaphoreType.DMA,
        pltpu.SemaphoreType.DMA,
        pltpu.SemaphoreType.DMA,
        pltpu.SemaphoreType.DMA,
    ],
    compiler_params=pltpu.CompilerParams(
        use_tc_tiling_on_sc=False, needs_layout_passes=False
    ),
)
def _gather(tbl, idxt, out, idx_v, rows_v, out_v,
            sg0, sg1, sg2, sg3, si0, si1, si2, si3, sw):
    _k2_body(tbl, idxt, out, idx_v, rows_v, out_v,
             sg0, sg1, sg2, sg3, si0, si1, si2, si3, sw)


def kernel(x, embedding_weight):
    wt = embedding_weight.T
    wtail = embedding_weight[_VMAIN:].T
    tbl2 = _repack(wt, wtail)
    tbl = tbl2.reshape(_V, _D)
    idxt = x.astype(jnp.int32).T
    o5 = _gather(tbl, idxt)
    return o5.transpose(2, 4, 0, 1, 3).reshape(_B, _L, _D)


# speedup vs baseline: 1.0741x; 1.0696x over previous
import functools
import math

import jax
import jax.numpy as jnp
from jax import lax
from jax.experimental import pallas as pl
from jax.experimental.pallas import tpu as pltpu
from jax.experimental.pallas import tpu_sc as plsc

_D = 64
_B = 4096
_L = 200
_V = 1000000
_N = _B * _L
_NC = 2
_NS = 16
_NW = _NC * _NS
_SCALE = math.sqrt(_D)

_VT = _V // 128
_VMAIN = _VT * 128
_TPB = 2
_QT = _VT // _TPB
_Q_PER_W = _QT // _NW
_Q_EXTRA = _QT - _Q_PER_W * _NW

_LPB = 2
_NBLK = _L // _LPB


def _iota16():
    return lax.iota(jnp.int32, 16)


def _k1_body(wt, wtail, tbl, in_v, out_v, tail_v, sg0, sg1, sw):
    wid = lax.axis_index("s") * _NC + lax.axis_index("c")
    start = wid * _Q_PER_W + jnp.minimum(wid, _Q_EXTRA)
    cnt = _Q_PER_W + jnp.where(wid < _Q_EXTRA, 1, 0)
    sg = (sg0, sg1)

    perm = [((g * 16 + _iota16()) % 64, (g * 16) // 64) for g in range(8)]

    def g_issue(q, b):
        for i in range(8):
            pltpu.async_copy(
                wt.at[pl.ds(i * 8, 8), pl.ds(q * (128 * _TPB), 128 * _TPB)],
                in_v.at[b, pl.ds(i * 8, 8), pl.ds(0, 128 * _TPB)],
                sg[b],
            )

    def g_wait(b):
        pltpu.make_async_copy(
            wt.at[:, pl.ds(0, 128 * _TPB)],
            in_v.at[b, :, pl.ds(0, 128 * _TPB)],
            sg[b],
        ).wait()

    def w_issue(q, b):
        pltpu.async_copy(
            out_v.at[b], tbl.at[pl.ds(q * (64 * _TPB), 64 * _TPB)], sw
        )

    def w_wait(b):
        pltpu.make_async_copy(
            tbl.at[pl.ds(0, 64 * _TPB)], out_v.at[b], sw
        ).wait()

    g_issue(start, 0)

    @pl.loop(0, (_Q_PER_W + 1 + 1) // 2 + 1)
    def _pairk(p):
        for b in range(2):
            k = p * 2 + b

            @pl.when(k < cnt)
            def _(b=b, k=k):
                q = start + k
                g_wait(b)

                @pl.when(k + 1 < cnt)
                def _():
                    g_issue(q + 1, 1 - b)

                @pl.when(k >= 2)
                def _():
                    w_wait(b)

                @plsc.parallel_loop(0, 64 * _TPB, 1, unroll=2)
                def _tp(pp, _b=b):
                    for g, (d0, gbit) in enumerate(perm):
                        vals = plsc.load_gather(
                            in_v.at[_b],
                            [d0, jnp.zeros((16,), jnp.int32) + (2 * pp + gbit)],
                        )
                        out_v[_b, pp, pl.ds(g * 16, 16)] = vals * _SCALE

                w_issue(q, b)

    for i in range(2):
        @pl.when(cnt > i)
        def _(i=i):
            w_wait((cnt - 2 + i) % 2)

    @pl.when(wid == _NW - 1)
    def _tail():
        pltpu.sync_copy(wtail, tail_v)

        @plsc.parallel_loop(0, 32, 1, unroll=2)
        def _tp(pp):
            for g, (d0, gbit) in enumerate(perm):
                vals = plsc.load_gather(
                    tail_v,
                    [d0, jnp.zeros((16,), jnp.int32) + (2 * pp + gbit)],
                )
                out_v[0, pp, pl.ds(g * 16, 16)] = vals * _SCALE

        pltpu.sync_copy(
            out_v.at[0, pl.ds(0, 32)], tbl.at[pl.ds(_VMAIN // 2, 32)]
        )


@functools.partial(
    pl.kernel,
    out_type=jax.ShapeDtypeStruct((_V // 2, 128), jnp.float32),
    mesh=plsc.VectorSubcoreMesh(core_axis_name="c", subcore_axis_name="s"),
    scratch_types=[
        pltpu.VMEM((2, _D, 128 * _TPB + 5), jnp.float32),
        pltpu.VMEM((2, 64 * _TPB, 128), jnp.float32),
        pltpu.VMEM((_D, _D), jnp.float32),
        pltpu.SemaphoreType.DMA,
        pltpu.SemaphoreType.DMA,
        pltpu.SemaphoreType.DMA,
    ],
    compiler_params=pltpu.CompilerParams(
        use_tc_tiling_on_sc=True, needs_layout_passes=False
    ),
)
def _repack(wt, wtail, tbl, in_v, out_v, tail_v, sg0, sg1, sw):
    _k1_body(wt, wtail, tbl, in_v, out_v, tail_v, sg0, sg1, sw)


def _k2_body(tbl, idxt, out, idx_v, rows_v, out_v,
             sg0, sg1, sg2, sg3, si0, si1, si2, si3, sw):
    wid = lax.axis_index("s") * _NC + lax.axis_index("c")
    sg = (sg0, sg1, sg2, sg3)
    si = (si0, si1, si2, si3)

    perm = []
    for u in range(_LPB):
        pu = []
        for j in range(4):
            c = j * 16 + _iota16()
            pu.append((u * 8 + c // 8, c % 8))
        perm.append(pu)

    def i_issue(blk, b):
        pltpu.async_copy(
            idxt.at[pl.ds(blk * _LPB, _LPB), pl.ds(wid * 128, 128)],
            idx_v.at[b], si[b],
        )

    def i_wait(blk, b):
        pltpu.make_async_copy(
            idxt.at[pl.ds(blk * _LPB, _LPB), pl.ds(wid * 128, 128)],
            idx_v.at[b], si[b],
        ).wait()

    def g_issue(b):
        for u in range(_LPB):
            pltpu.async_copy(
                tbl.at[idx_v.at[b, u]],
                rows_v.at[b, pl.ds(u * 128, 128)],
                sg[b],
            )

    def g_wait(b):
        pltpu.make_async_copy(tbl.at[pl.ds(0, _LPB * 128)], rows_v.at[b], sg[b]).wait()

    def w_issue(blk, b):
        l0 = blk * _LPB
        for u in range(_LPB):
            pltpu.async_copy(
                out_v.at[b, pl.ds(u * 8, 8), :, pl.ds(0, 128)],
                out.at[l0 + u, :, wid],
                sw,
            )

    def w_wait(b):
        for u in range(_LPB):
            pltpu.make_async_copy(
                out.at[0, :, 0],
                out_v.at[b, pl.ds(u * 8, 8), :, pl.ds(0, 128)],
                sw,
            ).wait()

    for i in range(3):
        i_issue(jnp.int32(i), i)
    i_wait(jnp.int32(0), 0)
    g_issue(0)
    i_wait(jnp.int32(1), 1)
    g_issue(1)

    @pl.loop(0, _NBLK // 4)
    def _quad(p):
        for b in range(4):
            blk = p * 4 + b

            g_wait(b)

            @pl.when(blk + 3 < _NBLK)
            def _(b=b, blk=blk):
                i_issue(blk + 3, (b + 3) % 4)

            @pl.when(blk + 2 < _NBLK)
            def _(b=b, blk=blk):
                i_wait(blk + 2, (b + 2) % 4)
                g_issue((b + 2) % 4)

            @pl.when(blk >= 2)
            def _(b=b):
                w_wait(b % 2)

            for u in range(_LPB):

                @plsc.parallel_loop(0, 128, 1, unroll=2)
                def _tp(rr, _b=b, _u=u):
                    for jj in range(4):
                        d0, d1 = perm[_u][jj]
                        vals = rows_v[_b, _u * 128 + rr, pl.ds(jj * 16, 16)]
                        plsc.store_scatter(
                            out_v.at[_b % 2],
                            [d0, d1, jnp.zeros((16,), jnp.int32) + rr],
                            vals,
                        )

            w_issue(blk, b % 2)

    @pl.loop(0, 1)
    def _fin(_):
        for blkf in (_NBLK - 2, _NBLK - 1):
            w_wait(blkf % 2)


@functools.partial(
    pl.kernel,
    out_type=jax.ShapeDtypeStruct((_L, 8, _NW, 8, 128), jnp.float32),
    mesh=plsc.VectorSubcoreMesh(core_axis_name="c", subcore_axis_name="s"),
    scratch_types=[
        pltpu.VMEM((4, _LPB, 128), jnp.int32),
        pltpu.VMEM((4, _LPB * 128, _D), jnp.float32),
        pltpu.VMEM((2, _LPB * 8, 8, 137), jnp.float32),
        pltpu.SemaphoreType.DMA,
        pltpu.SemaphoreType.DMA,
        pltpu.SemaphoreType.DMA,
        pltpu.SemaphoreType.DMA,
        pltpu.SemaphoreType.DMA,
        pltpu.SemaphoreType.DMA,
        pltpu.SemaphoreType.DMA,
        pltpu.SemaphoreType.DMA,
        pltpu.SemaphoreType.DMA,
    ],
    compiler_params=pltpu.CompilerParams(
        use_tc_tiling_on_sc=False, needs_layout_passes=False
    ),
)
def _gather(tbl, idxt, out, idx_v, rows_v, out_v,
            sg0, sg1, sg2, sg3, si0, si1, si2, si3, sw):
    _k2_body(tbl, idxt, out, idx_v, rows_v, out_v,
             sg0, sg1, sg2, sg3, si0, si1, si2, si3, sw)


def kernel(x, embedding_weight):
    wt = embedding_weight.T
    wtail = embedding_weight[_VMAIN:].T
    tbl2 = _repack(wt, wtail)
    tbl = tbl2.reshape(_V, _D)
    idxt = x.astype(jnp.int32).T
    o5 = _gather(tbl, idxt)
    return o5.transpose(2, 4, 0, 1, 3).reshape(_B, _L, _D)
